# baseline (speedup 1.0000x reference)
import functools

import jax
import jax.numpy as jnp
from jax import lax
from jax.experimental import pallas as pl
from jax.experimental.pallas import tpu as pltpu
from jax.experimental.pallas import tpu_sc as plsc

N_NODES = 10000
N_EDGES = 320000
N_PAD = 10240
E_PAD = 327680
EPT = E_PAD // 32
G = 512
NG = EPT // G
RPT = N_PAD // 16

_MESH = plsc.VectorSubcoreMesh(core_axis_name="c", subcore_axis_name="s")
_SC_PARAMS = pltpu.CompilerParams(
    use_tc_tiling_on_sc=False, needs_layout_passes=False)


def _rsqrt16(d):
    i = plsc.bitcast(d, jnp.int32)
    y = plsc.bitcast(0x5F3759DF - lax.shift_right_logical(i, 1), jnp.float32)
    for _ in range(3):
        y = y * (1.5 - 0.5 * d * y * y)
    return y


def _zero_rows(ref, n):
    z = jnp.zeros((16,), jnp.float32)

    def body(i, _):
        ref[i, :] = z
        return 0

    lax.fori_loop(0, n, body, 0, unroll=8)


@functools.partial(
    pl.kernel,
    out_type=jax.ShapeDtypeStruct((2, N_PAD), jnp.float32),
    mesh=_MESH,
    scratch_types=[
        pltpu.VMEM((EPT,), jnp.int32),
        pltpu.VMEM((EPT,), jnp.float32),
        pltpu.VMEM((RPT,), jnp.float32),
        pltpu.VMEM_SHARED((N_PAD,), jnp.float32),
    ],
    compiler_params=_SC_PARAMS,
)
def _sc_degree(dst_hbm, out_hbm, dst_v, ones_v, buf_v, deg_sh):
    c = lax.axis_index("c")
    s = lax.axis_index("s")

    one = jnp.ones((16,), jnp.float32)
    z = jnp.zeros((16,), jnp.float32)

    def ob(i, _):
        ones_v[pl.ds(i * 16, 16)] = one
        return 0

    lax.fori_loop(0, EPT // 16, ob, 0, unroll=8)

    def zb(i, _):
        buf_v[pl.ds(i * 16, 16)] = z
        return 0

    lax.fori_loop(0, RPT // 16, zb, 0, unroll=8)
    pltpu.sync_copy(buf_v, deg_sh.at[pl.ds(s * RPT, RPT)])
    plsc.subcore_barrier()

    for half in range(2):
        pltpu.sync_copy(dst_hbm.at[half * 16 + s], dst_v)
        pltpu.sync_copy(ones_v, deg_sh.at[dst_v], add=True)
    plsc.subcore_barrier()
    pltpu.sync_copy(deg_sh.at[pl.ds(s * RPT, RPT)], buf_v)
    pltpu.sync_copy(buf_v, out_hbm.at[c, pl.ds(s * RPT, RPT)])


def _propagate(w, src_hbm, dst_hbm, table_sh, acc_sh,
               sa, da, ra, gsa, isa, sb, db, rb, gsb, isb):
    def load_idx(g, srcb, dstb, isem):
        pltpu.async_copy(src_hbm.at[w, pl.ds(g * G, G)], srcb, isem)
        pltpu.async_copy(dst_hbm.at[w, pl.ds(g * G, G)], dstb, isem)

    def wait_idx(srcb, dstb, isem):
        pltpu.make_async_copy(src_hbm.at[w, pl.ds(0, G)], srcb, isem).wait()
        pltpu.make_async_copy(dst_hbm.at[w, pl.ds(0, G)], dstb, isem).wait()

    def wait_gather(rows, gsem):
        pltpu.make_async_copy(table_sh.at[sa], rows, gsem).wait()

    load_idx(0, sa, da, isa)
    wait_idx(sa, da, isa)
    pltpu.async_copy(table_sh.at[sa], ra, gsa)
    load_idx(1, sb, db, isb)

    def pair(p, _):
        wait_idx(sb, db, isb)
        pltpu.async_copy(table_sh.at[sb], rb, gsb)
        wait_gather(ra, gsa)
        pltpu.sync_copy(ra, acc_sh.at[da], add=True)
        load_idx(2 * p + 2, sa, da, isa)
        wait_gather(rb, gsb)
        pltpu.sync_copy(rb, acc_sh.at[db], add=True)
        load_idx(2 * p + 3, sb, db, isb)
        wait_idx(sa, da, isa)
        pltpu.async_copy(table_sh.at[sa], ra, gsa)
        return 0

    lax.fori_loop(0, NG // 2 - 1, pair, 0)
    wait_idx(sb, db, isb)
    pltpu.async_copy(table_sh.at[sb], rb, gsb)
    wait_gather(ra, gsa)
    pltpu.sync_copy(ra, acc_sh.at[da], add=True)
    wait_gather(rb, gsb)
    pltpu.sync_copy(rb, acc_sh.at[db], add=True)


_PROP_SCRATCH = [
    pltpu.VMEM((G,), jnp.int32),
    pltpu.VMEM((G,), jnp.int32),
    pltpu.VMEM((G, 16), jnp.float32),
    pltpu.SemaphoreType.DMA,
    pltpu.SemaphoreType.DMA,
    pltpu.VMEM((G,), jnp.int32),
    pltpu.VMEM((G,), jnp.int32),
    pltpu.VMEM((G, 16), jnp.float32),
    pltpu.SemaphoreType.DMA,
    pltpu.SemaphoreType.DMA,
]


@functools.partial(
    pl.kernel,
    out_type=(
        jax.ShapeDtypeStruct((2, N_PAD, 16), jnp.float32),
        jax.ShapeDtypeStruct((N_PAD,), jnp.float32),
        jax.ShapeDtypeStruct((2, N_PAD, 16), jnp.float32),
    ),
    mesh=_MESH,
    scratch_types=[
        pltpu.VMEM((RPT,), jnp.float32),
        pltpu.VMEM((RPT,), jnp.float32),
        pltpu.VMEM((RPT, 16), jnp.float32),
        pltpu.VMEM_SHARED((N_PAD, 16), jnp.float32),
    ] + _PROP_SCRATCH,
    compiler_params=_SC_PARAMS,
)
def _sc_layer1(src_hbm, dst_hbm, h1_hbm, deg_hbm, acc_out, dinv_out, tab_out,
               deg_v, dinv_v, h1_v, acc_sh,
               sa, da, ra, gsa, isa, sb, db, rb, gsb, isb):
    c = lax.axis_index("c")
    s = lax.axis_index("s")
    w = c * 16 + s

    sl = pl.ds(s * RPT, RPT)
    pltpu.sync_copy(deg_hbm.at[c, sl], deg_v)
    pltpu.sync_copy(h1_hbm.at[sl], h1_v)

    def dg(i, _):
        d = deg_v[pl.ds(i * 16, 16)] + 1.0
        dinv_v[pl.ds(i * 16, 16)] = _rsqrt16(d)
        return 0

    lax.fori_loop(0, RPT // 16, dg, 0)

    def rscale(g, _):
        dv = dinv_v[pl.ds(g * 16, 16)]
        for j in range(16):
            r = g * 16 + j
            h1_v[r, :] = h1_v[r, :] * dv[j]
        return 0

    lax.fori_loop(0, RPT // 16, rscale, 0)
    pltpu.sync_copy(h1_v, tab_out.at[c, sl])

    @pl.when(c == 0)
    def _():
        pltpu.sync_copy(h1_v, acc_sh.at[sl])
        pltpu.sync_copy(dinv_v, dinv_out.at[sl])

    @pl.when(c == 1)
    def _():
        _zero_rows(h1_v, RPT)
        pltpu.sync_copy(h1_v, acc_sh.at[sl])

    plsc.subcore_barrier()
    _propagate(w, src_hbm, dst_hbm, tab_out.at[c], acc_sh,
               sa, da, ra, gsa, isa, sb, db, rb, gsb, isb)
    plsc.subcore_barrier()
    for p in range(RPT // 128):
        sl2 = pl.ds(s * RPT + p * 128, 128)
        rp = ra.at[pl.ds(0, 128)]
        pltpu.sync_copy(acc_sh.at[sl2], rp)
        pltpu.sync_copy(rp, acc_out.at[c, sl2])


@functools.partial(
    pl.kernel,
    out_type=(
        jax.ShapeDtypeStruct((2, 16, N_PAD), jnp.float32),
        jax.ShapeDtypeStruct((2, N_PAD, 16), jnp.float32),
    ),
    mesh=_MESH,
    scratch_types=[
        pltpu.VMEM((RPT, 16), jnp.float32),
        pltpu.VMEM((RPT, 16), jnp.float32),
        pltpu.VMEM((RPT,), jnp.float32),
        pltpu.VMEM((16,), jnp.float32),
        pltpu.VMEM((16, 128), jnp.float32),
        pltpu.VMEM_SHARED((N_PAD, 16), jnp.float32),
    ] + _PROP_SCRATCH,
    compiler_params=_SC_PARAMS,
)
def _sc_layer2(src_hbm, dst_hbm, acc1_hbm, dinv_hbm, b1_hbm, acc_out, tab_out,
               a0_v, a1_v, dinv_v, b1_v, t_v, acc_sh,
               sa, da, ra, gsa, isa, sb, db, rb, gsb, isb):
    c = lax.axis_index("c")
    s = lax.axis_index("s")
    w = c * 16 + s

    sl = pl.ds(s * RPT, RPT)
    pltpu.sync_copy(acc1_hbm.at[0, sl], a0_v)
    pltpu.sync_copy(acc1_hbm.at[1, sl], a1_v)
    pltpu.sync_copy(dinv_hbm.at[sl], dinv_v)
    pltpu.sync_copy(b1_hbm, b1_v)
    b1 = b1_v[...]

    def r2row(g, _):
        dv = dinv_v[pl.ds(g * 16, 16)]
        for j in range(16):
            r = g * 16 + j
            t = dv[j] * (a0_v[r, :] + a1_v[r, :]) + b1
            a0_v[r, :] = dv[j] * jnp.maximum(t, 0.0)
        return 0

    lax.fori_loop(0, RPT // 16, r2row, 0)
    pltpu.sync_copy(a0_v, tab_out.at[c, sl])

    @pl.when(c == 0)
    def _():
        pltpu.sync_copy(a0_v, acc_sh.at[sl])

    @pl.when(c == 1)
    def _():
        _zero_rows(a0_v, RPT)
        pltpu.sync_copy(a0_v, acc_sh.at[sl])

    plsc.subcore_barrier()
    _propagate(w, src_hbm, dst_hbm, tab_out.at[c], acc_sh,
               sa, da, ra, gsa, isa, sb, db, rb, gsb, isb)
    plsc.subcore_barrier()

    lanes = lax.iota(jnp.int32, 16)
    rp = ra.at[pl.ds(0, 128)]
    rq = rb.at[pl.ds(0, 128)]
    pltpu.async_copy(acc_sh.at[pl.ds(s * RPT, 128)], rp, gsa)
    for p in range(RPT // 128):
        buf, nbuf, sem, nsem = ((ra, rb, gsa, gsb) if p % 2 == 0
                                else (rb, ra, gsb, gsa))
        pltpu.make_async_copy(
            acc_sh.at[pl.ds(s * RPT, 128)],
            rp if p % 2 == 0 else rq, sem).wait()
        if p + 1 < RPT // 128:
            pltpu.async_copy(acc_sh.at[pl.ds(s * RPT + (p + 1) * 128, 128)],
                             rq if p % 2 == 0 else rp, nsem)

        def tb(r, _):
            v = buf[r, :]
            plsc.store_scatter(t_v, [lanes, jnp.full((16,), r, jnp.int32)], v)
            return 0

        lax.fori_loop(0, 128, tb, 0, unroll=8)
        pltpu.sync_copy(
            t_v, acc_out.at[c, :, pl.ds(s * RPT + p * 128, 128)])


def _tc_mm1_body(x_ref, w1_ref, out_ref):
    out_ref[...] = jnp.dot(x_ref[...], w1_ref[...],
                           preferred_element_type=jnp.float32)


def _tc_mm1(x, w1):
    return pl.pallas_call(
        _tc_mm1_body,
        out_shape=jax.ShapeDtypeStruct((N_NODES, 16), jnp.float32),
    )(x, w1)


def _tc_out_body(acc_ref, dinv_ref, w2_ref, b2_ref, out_ref):
    a = (acc_ref[0] + acc_ref[1]) * dinv_ref[...][None, :]
    z = lax.dot_general(w2_ref[...], a, (((0,), (0,)), ((), ())),
                        preferred_element_type=jnp.float32)
    z = z + b2_ref[...][:, None]
    m = jnp.max(z, axis=0, keepdims=True)
    t = z - m
    out_ref[...] = t - jnp.log(jnp.sum(jnp.exp(t), axis=0, keepdims=True))


def _tc_out(acc, dinv, w2, b2):
    return pl.pallas_call(
        _tc_out_body,
        out_shape=jax.ShapeDtypeStruct((7, N_PAD), jnp.float32),
    )(acc, dinv, w2, b2)


def kernel(x, edge_index, W1, b1, W2, b2):
    ei = edge_index.astype(jnp.int32)
    src, dst = ei[0], ei[1]
    npad = E_PAD - N_EDGES
    pad_idx = N_NODES + jnp.arange(npad, dtype=jnp.int32) % (N_PAD - N_NODES)
    srcp = jnp.concatenate([src, pad_idx]).reshape(32, EPT)
    dstp = jnp.concatenate([dst, pad_idx]).reshape(32, EPT)

    deg = _sc_degree(dstp)
    h1 = _tc_mm1(x, W1)
    h1_pad = jnp.pad(h1, ((0, N_PAD - N_NODES), (0, 0)))
    acc1, dinv, _ = _sc_layer1(srcp, dstp, h1_pad, deg)
    acc2, _ = _sc_layer2(srcp, dstp, acc1, dinv, b1)
    zt = _tc_out(acc2, dinv, W2, b2)
    return zt[:, :N_NODES].T


# speedup vs baseline: 1.0619x; 1.0619x over previous
import functools

import jax
import jax.numpy as jnp
from jax import lax
from jax.experimental import pallas as pl
from jax.experimental.pallas import tpu as pltpu
from jax.experimental.pallas import tpu_sc as plsc

N_NODES = 10000
N_EDGES = 320000
N_PAD = 10240
E_PAD = 327680
EPT = E_PAD // 32
G = 512
NG = EPT // G
RPT = N_PAD // 16

_MESH = plsc.VectorSubcoreMesh(core_axis_name="c", subcore_axis_name="s")
_SC_PARAMS = pltpu.CompilerParams(
    use_tc_tiling_on_sc=False, needs_layout_passes=False)


def _rsqrt16(d):
    i = plsc.bitcast(d, jnp.int32)
    y = plsc.bitcast(0x5F3759DF - lax.shift_right_logical(i, 1), jnp.float32)
    for _ in range(3):
        y = y * (1.5 - 0.5 * d * y * y)
    return y


def _zero_rows(ref, n):
    z = jnp.zeros((16,), jnp.float32)

    def body(i, _):
        ref[i, :] = z
        return 0

    lax.fori_loop(0, n, body, 0, unroll=8)


@functools.partial(
    pl.kernel,
    out_type=jax.ShapeDtypeStruct((2, N_PAD), jnp.float32),
    mesh=_MESH,
    scratch_types=[
        pltpu.VMEM((EPT,), jnp.int32),
        pltpu.VMEM((EPT,), jnp.float32),
        pltpu.VMEM((RPT,), jnp.float32),
        pltpu.VMEM_SHARED((N_PAD,), jnp.float32),
    ],
    compiler_params=_SC_PARAMS,
)
def _sc_degree(dst_hbm, out_hbm, dst_v, ones_v, buf_v, deg_sh):
    c = lax.axis_index("c")
    s = lax.axis_index("s")

    one = jnp.ones((16,), jnp.float32)
    z = jnp.zeros((16,), jnp.float32)

    def ob(i, _):
        ones_v[pl.ds(i * 16, 16)] = one
        return 0

    lax.fori_loop(0, EPT // 16, ob, 0, unroll=8)

    def zb(i, _):
        buf_v[pl.ds(i * 16, 16)] = z
        return 0

    lax.fori_loop(0, RPT // 16, zb, 0, unroll=8)
    pltpu.sync_copy(buf_v, deg_sh.at[pl.ds(s * RPT, RPT)])
    plsc.subcore_barrier()

    for half in range(2):
        pltpu.sync_copy(dst_hbm.at[half * 16 + s], dst_v)
        pltpu.sync_copy(ones_v, deg_sh.at[dst_v], add=True)
    plsc.subcore_barrier()
    pltpu.sync_copy(deg_sh.at[pl.ds(s * RPT, RPT)], buf_v)
    pltpu.sync_copy(buf_v, out_hbm.at[c, pl.ds(s * RPT, RPT)])


def _propagate(w, src_hbm, dst_hbm, table_sh, acc_sh,
               sa, da, ra, gsa, isa, sb, db, rb, gsb, isb):
    def load_idx(g, srcb, dstb, isem):
        pltpu.async_copy(src_hbm.at[w, pl.ds(g * G, G)], srcb, isem)
        pltpu.async_copy(dst_hbm.at[w, pl.ds(g * G, G)], dstb, isem)

    def wait_idx(srcb, dstb, isem):
        pltpu.make_async_copy(src_hbm.at[w, pl.ds(0, G)], srcb, isem).wait()
        pltpu.make_async_copy(dst_hbm.at[w, pl.ds(0, G)], dstb, isem).wait()

    def wait_gather(rows, gsem):
        pltpu.make_async_copy(table_sh.at[sa], rows, gsem).wait()

    load_idx(0, sa, da, isa)
    wait_idx(sa, da, isa)
    pltpu.async_copy(table_sh.at[sa], ra, gsa)
    load_idx(1, sb, db, isb)

    def pair(p, _):
        wait_idx(sb, db, isb)
        pltpu.async_copy(table_sh.at[sb], rb, gsb)
        wait_gather(ra, gsa)
        pltpu.sync_copy(ra, acc_sh.at[da], add=True)
        load_idx(2 * p + 2, sa, da, isa)
        wait_gather(rb, gsb)
        pltpu.sync_copy(rb, acc_sh.at[db], add=True)
        load_idx(2 * p + 3, sb, db, isb)
        wait_idx(sa, da, isa)
        pltpu.async_copy(table_sh.at[sa], ra, gsa)
        return 0

    lax.fori_loop(0, NG // 2 - 1, pair, 0)
    wait_idx(sb, db, isb)
    pltpu.async_copy(table_sh.at[sb], rb, gsb)
    wait_gather(ra, gsa)
    pltpu.sync_copy(ra, acc_sh.at[da], add=True)
    wait_gather(rb, gsb)
    pltpu.sync_copy(rb, acc_sh.at[db], add=True)


_PROP_SCRATCH = [
    pltpu.VMEM((G,), jnp.int32),
    pltpu.VMEM((G,), jnp.int32),
    pltpu.VMEM((G, 16), jnp.float32),
    pltpu.SemaphoreType.DMA,
    pltpu.SemaphoreType.DMA,
    pltpu.VMEM((G,), jnp.int32),
    pltpu.VMEM((G,), jnp.int32),
    pltpu.VMEM((G, 16), jnp.float32),
    pltpu.SemaphoreType.DMA,
    pltpu.SemaphoreType.DMA,
]


@functools.partial(
    pl.kernel,
    out_type=(
        jax.ShapeDtypeStruct((2, N_PAD, 16), jnp.float32),
        jax.ShapeDtypeStruct((N_PAD,), jnp.float32),
    ),
    mesh=_MESH,
    scratch_types=[
        pltpu.VMEM((RPT,), jnp.float32),
        pltpu.VMEM((RPT,), jnp.float32),
        pltpu.VMEM((RPT, 16), jnp.float32),
        pltpu.VMEM_SHARED((N_PAD, 16), jnp.float32),
        pltpu.VMEM_SHARED((N_PAD, 16), jnp.float32),
    ] + _PROP_SCRATCH,
    compiler_params=_SC_PARAMS,
)
def _sc_layer1(src_hbm, dst_hbm, h1_hbm, deg_hbm, acc_out, dinv_out,
               deg_v, dinv_v, h1_v, table_sh, acc_sh,
               sa, da, ra, gsa, isa, sb, db, rb, gsb, isb):
    c = lax.axis_index("c")
    s = lax.axis_index("s")
    w = c * 16 + s

    sl = pl.ds(s * RPT, RPT)
    pltpu.sync_copy(deg_hbm.at[c, sl], deg_v)
    pltpu.sync_copy(h1_hbm.at[sl], h1_v)

    def dg(i, _):
        d = deg_v[pl.ds(i * 16, 16)] + 1.0
        dinv_v[pl.ds(i * 16, 16)] = _rsqrt16(d)
        return 0

    lax.fori_loop(0, RPT // 16, dg, 0)

    def rscale(g, _):
        dv = dinv_v[pl.ds(g * 16, 16)]
        for j in range(16):
            r = g * 16 + j
            h1_v[r, :] = h1_v[r, :] * dv[j]
        return 0

    lax.fori_loop(0, RPT // 16, rscale, 0)
    pltpu.sync_copy(h1_v, table_sh.at[sl])

    @pl.when(c == 0)
    def _():
        pltpu.sync_copy(h1_v, acc_sh.at[sl])
        pltpu.sync_copy(dinv_v, dinv_out.at[sl])

    @pl.when(c == 1)
    def _():
        _zero_rows(h1_v, RPT)
        pltpu.sync_copy(h1_v, acc_sh.at[sl])

    plsc.subcore_barrier()
    _propagate(w, src_hbm, dst_hbm, table_sh, acc_sh,
               sa, da, ra, gsa, isa, sb, db, rb, gsb, isb)
    plsc.subcore_barrier()
    for p in range(RPT // 128):
        sl2 = pl.ds(s * RPT + p * 128, 128)
        rp = ra.at[pl.ds(0, 128)]
        pltpu.sync_copy(acc_sh.at[sl2], rp)
        pltpu.sync_copy(rp, acc_out.at[c, sl2])


@functools.partial(
    pl.kernel,
    out_type=jax.ShapeDtypeStruct((2, 16, N_PAD), jnp.float32),
    mesh=_MESH,
    scratch_types=[
        pltpu.VMEM((RPT, 16), jnp.float32),
        pltpu.VMEM((RPT, 16), jnp.float32),
        pltpu.VMEM((RPT,), jnp.float32),
        pltpu.VMEM((16,), jnp.float32),
        pltpu.VMEM((16, 128), jnp.float32),
        pltpu.VMEM_SHARED((N_PAD, 16), jnp.float32),
        pltpu.VMEM_SHARED((N_PAD, 16), jnp.float32),
    ] + _PROP_SCRATCH,
    compiler_params=_SC_PARAMS,
)
def _sc_layer2(src_hbm, dst_hbm, acc1_hbm, dinv_hbm, b1_hbm, acc_out,
               a0_v, a1_v, dinv_v, b1_v, t_v, table_sh, acc_sh,
               sa, da, ra, gsa, isa, sb, db, rb, gsb, isb):
    c = lax.axis_index("c")
    s = lax.axis_index("s")
    w = c * 16 + s

    sl = pl.ds(s * RPT, RPT)
    pltpu.sync_copy(acc1_hbm.at[0, sl], a0_v)
    pltpu.sync_copy(acc1_hbm.at[1, sl], a1_v)
    pltpu.sync_copy(dinv_hbm.at[sl], dinv_v)
    pltpu.sync_copy(b1_hbm, b1_v)
    b1 = b1_v[...]

    def r2row(g, _):
        dv = dinv_v[pl.ds(g * 16, 16)]
        for j in range(16):
            r = g * 16 + j
            t = dv[j] * (a0_v[r, :] + a1_v[r, :]) + b1
            a0_v[r, :] = dv[j] * jnp.maximum(t, 0.0)
        return 0

    lax.fori_loop(0, RPT // 16, r2row, 0)
    pltpu.sync_copy(a0_v, table_sh.at[sl])

    @pl.when(c == 0)
    def _():
        pltpu.sync_copy(a0_v, acc_sh.at[sl])

    @pl.when(c == 1)
    def _():
        _zero_rows(a0_v, RPT)
        pltpu.sync_copy(a0_v, acc_sh.at[sl])

    plsc.subcore_barrier()
    _propagate(w, src_hbm, dst_hbm, table_sh, acc_sh,
               sa, da, ra, gsa, isa, sb, db, rb, gsb, isb)
    plsc.subcore_barrier()

    lanes = lax.iota(jnp.int32, 16)
    rp = ra.at[pl.ds(0, 128)]
    rq = rb.at[pl.ds(0, 128)]
    pltpu.async_copy(acc_sh.at[pl.ds(s * RPT, 128)], rp, gsa)
    for p in range(RPT // 128):
        buf, nbuf, sem, nsem = ((ra, rb, gsa, gsb) if p % 2 == 0
                                else (rb, ra, gsb, gsa))
        pltpu.make_async_copy(
            acc_sh.at[pl.ds(s * RPT, 128)],
            rp if p % 2 == 0 else rq, sem).wait()
        if p + 1 < RPT // 128:
            pltpu.async_copy(acc_sh.at[pl.ds(s * RPT + (p + 1) * 128, 128)],
                             rq if p % 2 == 0 else rp, nsem)

        def tb(r, _):
            v = buf[r, :]
            plsc.store_scatter(t_v, [lanes, jnp.full((16,), r, jnp.int32)], v)
            return 0

        lax.fori_loop(0, 128, tb, 0, unroll=8)
        pltpu.sync_copy(
            t_v, acc_out.at[c, :, pl.ds(s * RPT + p * 128, 128)])


def _tc_mm1_body(x_ref, w1_ref, out_ref):
    out_ref[...] = jnp.dot(x_ref[...], w1_ref[...],
                           preferred_element_type=jnp.float32)


def _tc_mm1(x, w1):
    return pl.pallas_call(
        _tc_mm1_body,
        out_shape=jax.ShapeDtypeStruct((N_NODES, 16), jnp.float32),
    )(x, w1)


def _tc_out_body(acc_ref, dinv_ref, w2_ref, b2_ref, out_ref):
    a = (acc_ref[0] + acc_ref[1]) * dinv_ref[...][None, :]
    z = lax.dot_general(w2_ref[...], a, (((0,), (0,)), ((), ())),
                        preferred_element_type=jnp.float32)
    z = z + b2_ref[...][:, None]
    m = jnp.max(z, axis=0, keepdims=True)
    t = z - m
    out_ref[...] = t - jnp.log(jnp.sum(jnp.exp(t), axis=0, keepdims=True))


def _tc_out(acc, dinv, w2, b2):
    return pl.pallas_call(
        _tc_out_body,
        out_shape=jax.ShapeDtypeStruct((7, N_PAD), jnp.float32),
    )(acc, dinv, w2, b2)


def kernel(x, edge_index, W1, b1, W2, b2):
    ei = edge_index.astype(jnp.int32)
    src, dst = ei[0], ei[1]
    npad = E_PAD - N_EDGES
    pad_idx = N_NODES + jnp.arange(npad, dtype=jnp.int32) % (N_PAD - N_NODES)
    srcp = jnp.concatenate([src, pad_idx]).reshape(32, EPT)
    dstp = jnp.concatenate([dst, pad_idx]).reshape(32, EPT)

    deg = _sc_degree(dstp)
    h1 = _tc_mm1(x, W1)
    h1_pad = jnp.pad(h1, ((0, N_PAD - N_NODES), (0, 0)))
    acc1, dinv = _sc_layer1(srcp, dstp, h1_pad, deg)
    acc2 = _sc_layer2(srcp, dstp, acc1, dinv, b1)
    zt = _tc_out(acc2, dinv, W2, b2)
    return zt[:, :N_NODES].T


# speedup vs baseline: 1.1014x; 1.0372x over previous
import functools

import jax
import jax.numpy as jnp
from jax import lax
from jax.experimental import pallas as pl
from jax.experimental.pallas import tpu as pltpu
from jax.experimental.pallas import tpu_sc as plsc

N_NODES = 10000
N_EDGES = 320000
N_PAD = 10240
E_PAD = 327680
EPT = E_PAD // 32
G = 1024
NG = EPT // G
RPT = N_PAD // 16

_MESH = plsc.VectorSubcoreMesh(core_axis_name="c", subcore_axis_name="s")
_SC_PARAMS = pltpu.CompilerParams(
    use_tc_tiling_on_sc=False, needs_layout_passes=False)


def _rsqrt16(d):
    i = plsc.bitcast(d, jnp.int32)
    y = plsc.bitcast(0x5F3759DF - lax.shift_right_logical(i, 1), jnp.float32)
    for _ in range(3):
        y = y * (1.5 - 0.5 * d * y * y)
    return y


def _zero_rows(ref, n):
    z = jnp.zeros((16,), jnp.float32)

    def body(i, _):
        ref[i, :] = z
        return 0

    lax.fori_loop(0, n, body, 0, unroll=8)


@functools.partial(
    pl.kernel,
    out_type=jax.ShapeDtypeStruct((2, N_PAD), jnp.float32),
    mesh=_MESH,
    scratch_types=[
        pltpu.VMEM((EPT,), jnp.int32),
        pltpu.VMEM((EPT,), jnp.float32),
        pltpu.VMEM((RPT,), jnp.float32),
        pltpu.VMEM_SHARED((N_PAD,), jnp.float32),
    ],
    compiler_params=_SC_PARAMS,
)
def _sc_degree(dst_hbm, out_hbm, dst_v, ones_v, buf_v, deg_sh):
    c = lax.axis_index("c")
    s = lax.axis_index("s")

    one = jnp.ones((16,), jnp.float32)
    z = jnp.zeros((16,), jnp.float32)

    def ob(i, _):
        ones_v[pl.ds(i * 16, 16)] = one
        return 0

    lax.fori_loop(0, EPT // 16, ob, 0, unroll=8)

    def zb(i, _):
        buf_v[pl.ds(i * 16, 16)] = z
        return 0

    lax.fori_loop(0, RPT // 16, zb, 0, unroll=8)
    pltpu.sync_copy(buf_v, deg_sh.at[pl.ds(s * RPT, RPT)])
    plsc.subcore_barrier()

    for half in range(2):
        pltpu.sync_copy(dst_hbm.at[half * 16 + s], dst_v)
        pltpu.sync_copy(ones_v, deg_sh.at[dst_v], add=True)
    plsc.subcore_barrier()
    pltpu.sync_copy(deg_sh.at[pl.ds(s * RPT, RPT)], buf_v)
    pltpu.sync_copy(buf_v, out_hbm.at[c, pl.ds(s * RPT, RPT)])


def _propagate(w, src_hbm, dst_hbm, table_sh, acc_sh,
               sa, da, ra, gsa, isa, sb, db, rb, gsb, isb):
    def load_idx(g, srcb, dstb, isem):
        pltpu.async_copy(src_hbm.at[w, pl.ds(g * G, G)], srcb, isem)
        pltpu.async_copy(dst_hbm.at[w, pl.ds(g * G, G)], dstb, isem)

    def wait_idx(srcb, dstb, isem):
        pltpu.make_async_copy(src_hbm.at[w, pl.ds(0, G)], srcb, isem).wait()
        pltpu.make_async_copy(dst_hbm.at[w, pl.ds(0, G)], dstb, isem).wait()

    def wait_gather(rows, gsem):
        pltpu.make_async_copy(table_sh.at[sa], rows, gsem).wait()

    load_idx(0, sa, da, isa)
    wait_idx(sa, da, isa)
    pltpu.async_copy(table_sh.at[sa], ra, gsa)
    load_idx(1, sb, db, isb)

    def pair(p, _):
        wait_idx(sb, db, isb)
        pltpu.async_copy(table_sh.at[sb], rb, gsb)
        wait_gather(ra, gsa)
        pltpu.sync_copy(ra, acc_sh.at[da], add=True)
        load_idx(2 * p + 2, sa, da, isa)
        wait_gather(rb, gsb)
        pltpu.sync_copy(rb, acc_sh.at[db], add=True)
        load_idx(2 * p + 3, sb, db, isb)
        wait_idx(sa, da, isa)
        pltpu.async_copy(table_sh.at[sa], ra, gsa)
        return 0

    lax.fori_loop(0, NG // 2 - 1, pair, 0)
    wait_idx(sb, db, isb)
    pltpu.async_copy(table_sh.at[sb], rb, gsb)
    wait_gather(ra, gsa)
    pltpu.sync_copy(ra, acc_sh.at[da], add=True)
    wait_gather(rb, gsb)
    pltpu.sync_copy(rb, acc_sh.at[db], add=True)


_PROP_SCRATCH = [
    pltpu.VMEM((G,), jnp.int32),
    pltpu.VMEM((G,), jnp.int32),
    pltpu.VMEM((G, 16), jnp.float32),
    pltpu.SemaphoreType.DMA,
    pltpu.SemaphoreType.DMA,
    pltpu.VMEM((G,), jnp.int32),
    pltpu.VMEM((G,), jnp.int32),
    pltpu.VMEM((G, 16), jnp.float32),
    pltpu.SemaphoreType.DMA,
    pltpu.SemaphoreType.DMA,
]


@functools.partial(
    pl.kernel,
    out_type=(
        jax.ShapeDtypeStruct((2, N_PAD, 16), jnp.float32),
        jax.ShapeDtypeStruct((N_PAD,), jnp.float32),
    ),
    mesh=_MESH,
    scratch_types=[
        pltpu.VMEM((RPT,), jnp.float32),
        pltpu.VMEM((RPT,), jnp.float32),
        pltpu.VMEM((RPT, 16), jnp.float32),
        pltpu.VMEM_SHARED((N_PAD, 16), jnp.float32),
        pltpu.VMEM_SHARED((N_PAD, 16), jnp.float32),
    ] + _PROP_SCRATCH,
    compiler_params=_SC_PARAMS,
)
def _sc_layer1(src_hbm, dst_hbm, h1_hbm, deg_hbm, acc_out, dinv_out,
               deg_v, dinv_v, h1_v, table_sh, acc_sh,
               sa, da, ra, gsa, isa, sb, db, rb, gsb, isb):
    c = lax.axis_index("c")
    s = lax.axis_index("s")
    w = c * 16 + s

    sl = pl.ds(s * RPT, RPT)
    pltpu.sync_copy(deg_hbm.at[c, sl], deg_v)
    pltpu.sync_copy(h1_hbm.at[sl], h1_v)

    def dg(i, _):
        d = deg_v[pl.ds(i * 16, 16)] + 1.0
        dinv_v[pl.ds(i * 16, 16)] = _rsqrt16(d)
        return 0

    lax.fori_loop(0, RPT // 16, dg, 0)

    def rscale(g, _):
        dv = dinv_v[pl.ds(g * 16, 16)]
        for j in range(16):
            r = g * 16 + j
            h1_v[r, :] = h1_v[r, :] * dv[j]
        return 0

    lax.fori_loop(0, RPT // 16, rscale, 0)
    pltpu.sync_copy(h1_v, table_sh.at[sl])

    @pl.when(c == 0)
    def _():
        pltpu.sync_copy(h1_v, acc_sh.at[sl])
        pltpu.sync_copy(dinv_v, dinv_out.at[sl])

    @pl.when(c == 1)
    def _():
        _zero_rows(h1_v, RPT)
        pltpu.sync_copy(h1_v, acc_sh.at[sl])

    plsc.subcore_barrier()
    _propagate(w, src_hbm, dst_hbm, table_sh, acc_sh,
               sa, da, ra, gsa, isa, sb, db, rb, gsb, isb)
    plsc.subcore_barrier()
    for p in range(RPT // 128):
        sl2 = pl.ds(s * RPT + p * 128, 128)
        rp = ra.at[pl.ds(0, 128)]
        pltpu.sync_copy(acc_sh.at[sl2], rp)
        pltpu.sync_copy(rp, acc_out.at[c, sl2])


@functools.partial(
    pl.kernel,
    out_type=jax.ShapeDtypeStruct((2, 16, N_PAD), jnp.float32),
    mesh=_MESH,
    scratch_types=[
        pltpu.VMEM((RPT, 16), jnp.float32),
        pltpu.VMEM((RPT, 16), jnp.float32),
        pltpu.VMEM((RPT,), jnp.float32),
        pltpu.VMEM((16,), jnp.float32),
        pltpu.VMEM((16, 128), jnp.float32),
        pltpu.VMEM_SHARED((N_PAD, 16), jnp.float32),
        pltpu.VMEM_SHARED((N_PAD, 16), jnp.float32),
    ] + _PROP_SCRATCH,
    compiler_params=_SC_PARAMS,
)
def _sc_layer2(src_hbm, dst_hbm, acc1_hbm, dinv_hbm, b1_hbm, acc_out,
               a0_v, a1_v, dinv_v, b1_v, t_v, table_sh, acc_sh,
               sa, da, ra, gsa, isa, sb, db, rb, gsb, isb):
    c = lax.axis_index("c")
    s = lax.axis_index("s")
    w = c * 16 + s

    sl = pl.ds(s * RPT, RPT)
    pltpu.sync_copy(acc1_hbm.at[0, sl], a0_v)
    pltpu.sync_copy(acc1_hbm.at[1, sl], a1_v)
    pltpu.sync_copy(dinv_hbm.at[sl], dinv_v)
    pltpu.sync_copy(b1_hbm, b1_v)
    b1 = b1_v[...]

    def r2row(g, _):
        dv = dinv_v[pl.ds(g * 16, 16)]
        for j in range(16):
            r = g * 16 + j
            t = dv[j] * (a0_v[r, :] + a1_v[r, :]) + b1
            a0_v[r, :] = dv[j] * jnp.maximum(t, 0.0)
        return 0

    lax.fori_loop(0, RPT // 16, r2row, 0)
    pltpu.sync_copy(a0_v, table_sh.at[sl])

    @pl.when(c == 0)
    def _():
        pltpu.sync_copy(a0_v, acc_sh.at[sl])

    @pl.when(c == 1)
    def _():
        _zero_rows(a0_v, RPT)
        pltpu.sync_copy(a0_v, acc_sh.at[sl])

    plsc.subcore_barrier()
    _propagate(w, src_hbm, dst_hbm, table_sh, acc_sh,
               sa, da, ra, gsa, isa, sb, db, rb, gsb, isb)
    plsc.subcore_barrier()

    lanes = lax.iota(jnp.int32, 16)
    rp = ra.at[pl.ds(0, 128)]
    rq = rb.at[pl.ds(0, 128)]
    pltpu.async_copy(acc_sh.at[pl.ds(s * RPT, 128)], rp, gsa)
    for p in range(RPT // 128):
        buf, nbuf, sem, nsem = ((ra, rb, gsa, gsb) if p % 2 == 0
                                else (rb, ra, gsb, gsa))
        pltpu.make_async_copy(
            acc_sh.at[pl.ds(s * RPT, 128)],
            rp if p % 2 == 0 else rq, sem).wait()
        if p + 1 < RPT // 128:
            pltpu.async_copy(acc_sh.at[pl.ds(s * RPT + (p + 1) * 128, 128)],
                             rq if p % 2 == 0 else rp, nsem)

        def tb(r, _):
            v = buf[r, :]
            plsc.store_scatter(t_v, [lanes, jnp.full((16,), r, jnp.int32)], v)
            return 0

        lax.fori_loop(0, 128, tb, 0, unroll=8)
        pltpu.sync_copy(
            t_v, acc_out.at[c, :, pl.ds(s * RPT + p * 128, 128)])


def _tc_mm1_body(x_ref, w1_ref, out_ref):
    out_ref[...] = jnp.dot(x_ref[...], w1_ref[...],
                           preferred_element_type=jnp.float32)


def _tc_mm1(x, w1):
    return pl.pallas_call(
        _tc_mm1_body,
        out_shape=jax.ShapeDtypeStruct((N_NODES, 16), jnp.float32),
    )(x, w1)


def _tc_out_body(acc_ref, dinv_ref, w2_ref, b2_ref, out_ref):
    a = (acc_ref[0] + acc_ref[1]) * dinv_ref[...][None, :]
    z = lax.dot_general(w2_ref[...], a, (((0,), (0,)), ((), ())),
                        preferred_element_type=jnp.float32)
    z = z + b2_ref[...][:, None]
    m = jnp.max(z, axis=0, keepdims=True)
    t = z - m
    out_ref[...] = t - jnp.log(jnp.sum(jnp.exp(t), axis=0, keepdims=True))


def _tc_out(acc, dinv, w2, b2):
    return pl.pallas_call(
        _tc_out_body,
        out_shape=jax.ShapeDtypeStruct((7, N_PAD), jnp.float32),
    )(acc, dinv, w2, b2)


def kernel(x, edge_index, W1, b1, W2, b2):
    ei = edge_index.astype(jnp.int32)
    src, dst = ei[0], ei[1]
    npad = E_PAD - N_EDGES
    pad_idx = N_NODES + jnp.arange(npad, dtype=jnp.int32) % (N_PAD - N_NODES)
    srcp = jnp.concatenate([src, pad_idx]).reshape(32, EPT)
    dstp = jnp.concatenate([dst, pad_idx]).reshape(32, EPT)

    deg = _sc_degree(dstp)
    h1 = _tc_mm1(x, W1)
    h1_pad = jnp.pad(h1, ((0, N_PAD - N_NODES), (0, 0)))
    acc1, dinv = _sc_layer1(srcp, dstp, h1_pad, deg)
    acc2 = _sc_layer2(srcp, dstp, acc1, dinv, b1)
    zt = _tc_out(acc2, dinv, W2, b2)
    return zt[:, :N_NODES].T


# speedup vs baseline: 1.1021x; 1.0007x over previous
import functools

import jax
import jax.numpy as jnp
from jax import lax
from jax.experimental import pallas as pl
from jax.experimental.pallas import tpu as pltpu
from jax.experimental.pallas import tpu_sc as plsc

N_NODES = 10000
N_EDGES = 320000
N_PAD = 10240
E_PAD = 327680
EPT = E_PAD // 32
G = 1024
NG = EPT // G
RPT = N_PAD // 16

_MESH = plsc.VectorSubcoreMesh(core_axis_name="c", subcore_axis_name="s")
_SC_PARAMS = pltpu.CompilerParams(
    use_tc_tiling_on_sc=False, needs_layout_passes=False)


def _rsqrt16(d):
    i = plsc.bitcast(d, jnp.int32)
    y = plsc.bitcast(0x5F3759DF - lax.shift_right_logical(i, 1), jnp.float32)
    for _ in range(3):
        y = y * (1.5 - 0.5 * d * y * y)
    return y


def _zero_rows(ref, n):
    z = jnp.zeros((16,), jnp.float32)

    def body(i, _):
        ref[i, :] = z
        return 0

    lax.fori_loop(0, n, body, 0, unroll=8)


@functools.partial(
    pl.kernel,
    out_type=jax.ShapeDtypeStruct((2, N_PAD), jnp.float32),
    mesh=_MESH,
    scratch_types=[
        pltpu.VMEM((EPT,), jnp.int32),
        pltpu.VMEM((EPT,), jnp.float32),
        pltpu.VMEM((RPT,), jnp.float32),
        pltpu.VMEM_SHARED((N_PAD,), jnp.float32),
    ],
    compiler_params=_SC_PARAMS,
)
def _sc_degree(dst_hbm, out_hbm, dst_v, ones_v, buf_v, deg_sh):
    c = lax.axis_index("c")
    s = lax.axis_index("s")

    one = jnp.ones((16,), jnp.float32)
    z = jnp.zeros((16,), jnp.float32)

    def ob(i, _):
        ones_v[pl.ds(i * 16, 16)] = one
        return 0

    lax.fori_loop(0, EPT // 16, ob, 0, unroll=8)

    def zb(i, _):
        buf_v[pl.ds(i * 16, 16)] = z
        return 0

    lax.fori_loop(0, RPT // 16, zb, 0, unroll=8)
    pltpu.sync_copy(buf_v, deg_sh.at[pl.ds(s * RPT, RPT)])
    plsc.subcore_barrier()

    for half in range(2):
        pltpu.sync_copy(dst_hbm.at[half * 16 + s], dst_v)
        pltpu.sync_copy(ones_v, deg_sh.at[dst_v], add=True)
    plsc.subcore_barrier()
    pltpu.sync_copy(deg_sh.at[pl.ds(s * RPT, RPT)], buf_v)
    pltpu.sync_copy(buf_v, out_hbm.at[c, pl.ds(s * RPT, RPT)])


def _propagate(w, src_hbm, dst_hbm, table_sh, acc_sh,
               sa, da, ra, gsa, isa, sb, db, rb, gsb, isb):
    def load_idx(g, srcb, dstb, isem):
        pltpu.async_copy(src_hbm.at[w, pl.ds(g * G, G)], srcb, isem)
        pltpu.async_copy(dst_hbm.at[w, pl.ds(g * G, G)], dstb, isem)

    def wait_idx(srcb, dstb, isem):
        pltpu.make_async_copy(src_hbm.at[w, pl.ds(0, G)], srcb, isem).wait()
        pltpu.make_async_copy(dst_hbm.at[w, pl.ds(0, G)], dstb, isem).wait()

    def wait_gather(rows, gsem):
        pltpu.make_async_copy(table_sh.at[sa], rows, gsem).wait()

    load_idx(0, sa, da, isa)
    wait_idx(sa, da, isa)
    pltpu.async_copy(table_sh.at[sa], ra, gsa)
    load_idx(1, sb, db, isb)

    def pair(p, _):
        wait_idx(sb, db, isb)
        pltpu.async_copy(table_sh.at[sb], rb, gsb)
        wait_gather(ra, gsa)
        pltpu.sync_copy(ra, acc_sh.at[da], add=True)
        load_idx(2 * p + 2, sa, da, isa)
        wait_gather(rb, gsb)
        pltpu.sync_copy(rb, acc_sh.at[db], add=True)
        load_idx(2 * p + 3, sb, db, isb)
        wait_idx(sa, da, isa)
        pltpu.async_copy(table_sh.at[sa], ra, gsa)
        return 0

    lax.fori_loop(0, NG // 2 - 1, pair, 0)
    wait_idx(sb, db, isb)
    pltpu.async_copy(table_sh.at[sb], rb, gsb)
    wait_gather(ra, gsa)
    pltpu.sync_copy(ra, acc_sh.at[da], add=True)
    wait_gather(rb, gsb)
    pltpu.sync_copy(rb, acc_sh.at[db], add=True)


_PROP_SCRATCH = [
    pltpu.VMEM((G,), jnp.int32),
    pltpu.VMEM((G,), jnp.int32),
    pltpu.VMEM((G, 16), jnp.float32),
    pltpu.SemaphoreType.DMA,
    pltpu.SemaphoreType.DMA,
    pltpu.VMEM((G,), jnp.int32),
    pltpu.VMEM((G,), jnp.int32),
    pltpu.VMEM((G, 16), jnp.float32),
    pltpu.SemaphoreType.DMA,
    pltpu.SemaphoreType.DMA,
]


@functools.partial(
    pl.kernel,
    out_type=(
        jax.ShapeDtypeStruct((2, N_PAD, 16), jnp.float32),
        jax.ShapeDtypeStruct((N_PAD,), jnp.float32),
    ),
    mesh=_MESH,
    scratch_types=[
        pltpu.VMEM((RPT,), jnp.float32),
        pltpu.VMEM((RPT,), jnp.float32),
        pltpu.VMEM((RPT, 16), jnp.float32),
        pltpu.VMEM_SHARED((N_PAD, 16), jnp.float32),
        pltpu.VMEM_SHARED((N_PAD, 16), jnp.float32),
    ] + _PROP_SCRATCH,
    compiler_params=_SC_PARAMS,
)
def _sc_layer1(src_hbm, dst_hbm, h1_hbm, deg_hbm, acc_out, dinv_out,
               deg_v, dinv_v, h1_v, table_sh, acc_sh,
               sa, da, ra, gsa, isa, sb, db, rb, gsb, isb):
    c = lax.axis_index("c")
    s = lax.axis_index("s")
    w = c * 16 + s

    sl = pl.ds(s * RPT, RPT)
    scope1 = jax.named_scope("l1_prolog"); scope1.__enter__()
    pltpu.sync_copy(deg_hbm.at[c, sl], deg_v)
    pltpu.sync_copy(h1_hbm.at[sl], h1_v)

    def dg(i, _):
        d = deg_v[pl.ds(i * 16, 16)] + 1.0
        dinv_v[pl.ds(i * 16, 16)] = _rsqrt16(d)
        return 0

    lax.fori_loop(0, RPT // 16, dg, 0)

    def rscale(g, _):
        dv = dinv_v[pl.ds(g * 16, 16)]
        for j in range(16):
            r = g * 16 + j
            h1_v[r, :] = h1_v[r, :] * dv[j]
        return 0

    lax.fori_loop(0, RPT // 16, rscale, 0)
    pltpu.sync_copy(h1_v, table_sh.at[sl])

    @pl.when(c == 0)
    def _():
        pltpu.sync_copy(h1_v, acc_sh.at[sl])
        pltpu.sync_copy(dinv_v, dinv_out.at[sl])

    @pl.when(c == 1)
    def _():
        _zero_rows(h1_v, RPT)
        pltpu.sync_copy(h1_v, acc_sh.at[sl])

    scope1.__exit__(None, None, None)
    plsc.subcore_barrier()
    with jax.named_scope("l1_prop"):
        _propagate(w, src_hbm, dst_hbm, table_sh, acc_sh,
                   sa, da, ra, gsa, isa, sb, db, rb, gsb, isb)
    plsc.subcore_barrier()
    for p in range(RPT // 128):
        sl2 = pl.ds(s * RPT + p * 128, 128)
        rp = ra.at[pl.ds(0, 128)]
        pltpu.sync_copy(acc_sh.at[sl2], rp)
        pltpu.sync_copy(rp, acc_out.at[c, sl2])


@functools.partial(
    pl.kernel,
    out_type=jax.ShapeDtypeStruct((2, 16, N_PAD), jnp.float32),
    mesh=_MESH,
    scratch_types=[
        pltpu.VMEM((RPT, 16), jnp.float32),
        pltpu.VMEM((RPT, 16), jnp.float32),
        pltpu.VMEM((RPT,), jnp.float32),
        pltpu.VMEM((16,), jnp.float32),
        pltpu.VMEM((16, 128), jnp.float32),
        pltpu.VMEM_SHARED((N_PAD, 16), jnp.float32),
        pltpu.VMEM_SHARED((N_PAD, 16), jnp.float32),
    ] + _PROP_SCRATCH,
    compiler_params=_SC_PARAMS,
)
def _sc_layer2(src_hbm, dst_hbm, acc1_hbm, dinv_hbm, b1_hbm, acc_out,
               a0_v, a1_v, dinv_v, b1_v, t_v, table_sh, acc_sh,
               sa, da, ra, gsa, isa, sb, db, rb, gsb, isb):
    c = lax.axis_index("c")
    s = lax.axis_index("s")
    w = c * 16 + s

    sl = pl.ds(s * RPT, RPT)
    scope2 = jax.named_scope("l2_prolog"); scope2.__enter__()
    pltpu.sync_copy(acc1_hbm.at[0, sl], a0_v)
    pltpu.sync_copy(acc1_hbm.at[1, sl], a1_v)
    pltpu.sync_copy(dinv_hbm.at[sl], dinv_v)
    pltpu.sync_copy(b1_hbm, b1_v)
    b1 = b1_v[...]

    def r2row(g, _):
        dv = dinv_v[pl.ds(g * 16, 16)]
        for j in range(16):
            r = g * 16 + j
            t = dv[j] * (a0_v[r, :] + a1_v[r, :]) + b1
            a0_v[r, :] = dv[j] * jnp.maximum(t, 0.0)
        return 0

    lax.fori_loop(0, RPT // 16, r2row, 0)
    pltpu.sync_copy(a0_v, table_sh.at[sl])

    @pl.when(c == 0)
    def _():
        pltpu.sync_copy(a0_v, acc_sh.at[sl])

    @pl.when(c == 1)
    def _():
        _zero_rows(a0_v, RPT)
        pltpu.sync_copy(a0_v, acc_sh.at[sl])

    scope2.__exit__(None, None, None)
    plsc.subcore_barrier()
    with jax.named_scope("l2_prop"):
        _propagate(w, src_hbm, dst_hbm, table_sh, acc_sh,
                   sa, da, ra, gsa, isa, sb, db, rb, gsb, isb)
    plsc.subcore_barrier()

    scope3 = jax.named_scope("l2_drain"); scope3.__enter__()
    lanes = lax.iota(jnp.int32, 16)
    rp = ra.at[pl.ds(0, 128)]
    rq = rb.at[pl.ds(0, 128)]
    pltpu.async_copy(acc_sh.at[pl.ds(s * RPT, 128)], rp, gsa)
    for p in range(RPT // 128):
        buf, nbuf, sem, nsem = ((ra, rb, gsa, gsb) if p % 2 == 0
                                else (rb, ra, gsb, gsa))
        pltpu.make_async_copy(
            acc_sh.at[pl.ds(s * RPT, 128)],
            rp if p % 2 == 0 else rq, sem).wait()
        if p + 1 < RPT // 128:
            pltpu.async_copy(acc_sh.at[pl.ds(s * RPT + (p + 1) * 128, 128)],
                             rq if p % 2 == 0 else rp, nsem)

        def tb(r, _):
            v = buf[r, :]
            plsc.store_scatter(t_v, [lanes, jnp.full((16,), r, jnp.int32)], v)
            return 0

        lax.fori_loop(0, 128, tb, 0, unroll=8)
        pltpu.sync_copy(
            t_v, acc_out.at[c, :, pl.ds(s * RPT + p * 128, 128)])
    scope3.__exit__(None, None, None)


def _tc_mm1_body(x_ref, w1_ref, out_ref):
    out_ref[...] = jnp.dot(x_ref[...], w1_ref[...],
                           preferred_element_type=jnp.float32)


def _tc_mm1(x, w1):
    return pl.pallas_call(
        _tc_mm1_body,
        out_shape=jax.ShapeDtypeStruct((N_NODES, 16), jnp.float32),
    )(x, w1)


def _tc_out_body(acc_ref, dinv_ref, w2_ref, b2_ref, out_ref):
    a = (acc_ref[0] + acc_ref[1]) * dinv_ref[...][None, :]
    z = lax.dot_general(w2_ref[...], a, (((0,), (0,)), ((), ())),
                        preferred_element_type=jnp.float32)
    z = z + b2_ref[...][:, None]
    m = jnp.max(z, axis=0, keepdims=True)
    t = z - m
    out_ref[...] = t - jnp.log(jnp.sum(jnp.exp(t), axis=0, keepdims=True))


def _tc_out(acc, dinv, w2, b2):
    return pl.pallas_call(
        _tc_out_body,
        out_shape=jax.ShapeDtypeStruct((7, N_PAD), jnp.float32),
    )(acc, dinv, w2, b2)


def kernel(x, edge_index, W1, b1, W2, b2):
    ei = edge_index.astype(jnp.int32)
    src, dst = ei[0], ei[1]
    npad = E_PAD - N_EDGES
    pad_idx = N_NODES + jnp.arange(npad, dtype=jnp.int32) % (N_PAD - N_NODES)
    srcp = jnp.concatenate([src, pad_idx]).reshape(32, EPT)
    dstp = jnp.concatenate([dst, pad_idx]).reshape(32, EPT)

    deg = _sc_degree(dstp)
    h1 = _tc_mm1(x, W1)
    h1_pad = jnp.pad(h1, ((0, N_PAD - N_NODES), (0, 0)))
    acc1, dinv = _sc_layer1(srcp, dstp, h1_pad, deg)
    acc2 = _sc_layer2(srcp, dstp, acc1, dinv, b1)
    zt = _tc_out(acc2, dinv, W2, b2)
    return zt[:, :N_NODES].T


# speedup vs baseline: 1.1048x; 1.0024x over previous
import functools

import jax
import jax.numpy as jnp
from jax import lax
from jax.experimental import pallas as pl
from jax.experimental.pallas import tpu as pltpu
from jax.experimental.pallas import tpu_sc as plsc

N_NODES = 10000
N_EDGES = 320000
N_PAD = 10240
E_PAD = 327680
EPT = E_PAD // 32
G = 1024
NG = EPT // G
RPT = N_PAD // 16

_MESH = plsc.VectorSubcoreMesh(core_axis_name="c", subcore_axis_name="s")
_SC_PARAMS = pltpu.CompilerParams(
    use_tc_tiling_on_sc=False, needs_layout_passes=False)


def _rsqrt16(d):
    i = plsc.bitcast(d, jnp.int32)
    y = plsc.bitcast(0x5F3759DF - lax.shift_right_logical(i, 1), jnp.float32)
    for _ in range(3):
        y = y * (1.5 - 0.5 * d * y * y)
    return y


def _zero_rows(ref, n):
    z = jnp.zeros((16,), jnp.float32)

    def body(i, _):
        ref[i, :] = z
        return 0

    lax.fori_loop(0, n, body, 0, unroll=8)


@functools.partial(
    pl.kernel,
    out_type=jax.ShapeDtypeStruct((2, N_PAD), jnp.float32),
    mesh=_MESH,
    scratch_types=[
        pltpu.VMEM((EPT,), jnp.int32),
        pltpu.VMEM((EPT,), jnp.float32),
        pltpu.VMEM((RPT,), jnp.float32),
        pltpu.VMEM_SHARED((N_PAD,), jnp.float32),
    ],
    compiler_params=_SC_PARAMS,
)
def _sc_degree(dst_hbm, out_hbm, dst_v, ones_v, buf_v, deg_sh):
    c = lax.axis_index("c")
    s = lax.axis_index("s")

    one = jnp.ones((16,), jnp.float32)
    z = jnp.zeros((16,), jnp.float32)

    def ob(i, _):
        ones_v[pl.ds(i * 16, 16)] = one
        return 0

    lax.fori_loop(0, EPT // 16, ob, 0, unroll=8)

    def zb(i, _):
        buf_v[pl.ds(i * 16, 16)] = z
        return 0

    lax.fori_loop(0, RPT // 16, zb, 0, unroll=8)
    pltpu.sync_copy(buf_v, deg_sh.at[pl.ds(s * RPT, RPT)])
    plsc.subcore_barrier()

    for half in range(2):
        pltpu.sync_copy(dst_hbm.at[half * 16 + s], dst_v)
        pltpu.sync_copy(ones_v, deg_sh.at[dst_v], add=True)
    plsc.subcore_barrier()
    pltpu.sync_copy(deg_sh.at[pl.ds(s * RPT, RPT)], buf_v)
    pltpu.sync_copy(buf_v, out_hbm.at[c, pl.ds(s * RPT, RPT)])


def _propagate(w, src_hbm, dst_hbm, table_sh, acc_sh,
               sa, da, ra, gsa, isa, sb, db, rb, gsb, isb):
    def load_idx(g, srcb, dstb, isem):
        pltpu.async_copy(src_hbm.at[w, pl.ds(g * G, G)], srcb, isem)
        pltpu.async_copy(dst_hbm.at[w, pl.ds(g * G, G)], dstb, isem)

    def wait_idx(srcb, dstb, isem):
        pltpu.make_async_copy(src_hbm.at[w, pl.ds(0, G)], srcb, isem).wait()
        pltpu.make_async_copy(dst_hbm.at[w, pl.ds(0, G)], dstb, isem).wait()

    def wait_gather(rows, gsem):
        pltpu.make_async_copy(table_sh.at[sa], rows, gsem).wait()

    load_idx(0, sa, da, isa)
    wait_idx(sa, da, isa)
    pltpu.async_copy(table_sh.at[sa], ra, gsa)
    load_idx(1, sb, db, isb)

    def pair(p, _):
        wait_idx(sb, db, isb)
        pltpu.async_copy(table_sh.at[sb], rb, gsb)
        wait_gather(ra, gsa)
        pltpu.sync_copy(ra, acc_sh.at[da], add=True)
        load_idx(2 * p + 2, sa, da, isa)
        wait_gather(rb, gsb)
        pltpu.sync_copy(rb, acc_sh.at[db], add=True)
        load_idx(2 * p + 3, sb, db, isb)
        wait_idx(sa, da, isa)
        pltpu.async_copy(table_sh.at[sa], ra, gsa)
        return 0

    lax.fori_loop(0, NG // 2 - 1, pair, 0)
    wait_idx(sb, db, isb)
    pltpu.async_copy(table_sh.at[sb], rb, gsb)
    wait_gather(ra, gsa)
    pltpu.sync_copy(ra, acc_sh.at[da], add=True)
    wait_gather(rb, gsb)
    pltpu.sync_copy(rb, acc_sh.at[db], add=True)


_PROP_SCRATCH = [
    pltpu.VMEM((G,), jnp.int32),
    pltpu.VMEM((G,), jnp.int32),
    pltpu.VMEM((G, 16), jnp.float32),
    pltpu.SemaphoreType.DMA,
    pltpu.SemaphoreType.DMA,
    pltpu.VMEM((G,), jnp.int32),
    pltpu.VMEM((G,), jnp.int32),
    pltpu.VMEM((G, 16), jnp.float32),
    pltpu.SemaphoreType.DMA,
    pltpu.SemaphoreType.DMA,
]


@functools.partial(
    pl.kernel,
    out_type=(
        jax.ShapeDtypeStruct((2, N_PAD, 16), jnp.float32),
        jax.ShapeDtypeStruct((N_PAD,), jnp.float32),
    ),
    mesh=_MESH,
    scratch_types=[
        pltpu.VMEM((RPT,), jnp.float32),
        pltpu.VMEM((RPT,), jnp.float32),
        pltpu.VMEM((RPT, 16), jnp.float32),
        pltpu.VMEM_SHARED((N_PAD, 16), jnp.float32),
        pltpu.VMEM_SHARED((N_PAD, 16), jnp.float32),
    ] + _PROP_SCRATCH,
    compiler_params=_SC_PARAMS,
)
def _sc_layer1(src_hbm, dst_hbm, h1_hbm, deg_hbm, acc_out, dinv_out,
               deg_v, dinv_v, h1_v, table_sh, acc_sh,
               sa, da, ra, gsa, isa, sb, db, rb, gsb, isb):
    c = lax.axis_index("c")
    s = lax.axis_index("s")
    w = c * 16 + s

    sl = pl.ds(s * RPT, RPT)
    pltpu.sync_copy(deg_hbm.at[c, sl], deg_v)
    pltpu.sync_copy(h1_hbm.at[sl], h1_v)

    def dg(i, _):
        d = deg_v[pl.ds(i * 16, 16)] + 1.0
        dinv_v[pl.ds(i * 16, 16)] = _rsqrt16(d)
        return 0

    lax.fori_loop(0, RPT // 16, dg, 0)

    def rscale(g, _):
        dv = dinv_v[pl.ds(g * 16, 16)]
        for j in range(16):
            r = g * 16 + j
            h1_v[r, :] = h1_v[r, :] * dv[j]
        return 0

    lax.fori_loop(0, RPT // 16, rscale, 0)
    pltpu.sync_copy(h1_v, table_sh.at[sl])

    @pl.when(c == 0)
    def _():
        pltpu.sync_copy(h1_v, acc_sh.at[sl])
        pltpu.sync_copy(dinv_v, dinv_out.at[sl])

    @pl.when(c == 1)
    def _():
        _zero_rows(h1_v, RPT)
        pltpu.sync_copy(h1_v, acc_sh.at[sl])

    plsc.subcore_barrier()
    _propagate(w, src_hbm, dst_hbm, table_sh, acc_sh,
               sa, da, ra, gsa, isa, sb, db, rb, gsb, isb)
    plsc.subcore_barrier()
    rp = ra.at[pl.ds(0, 128)]
    rq = rb.at[pl.ds(0, 128)]
    pltpu.sync_copy(acc_sh.at[pl.ds(s * RPT, 128)], rp)
    for p in range(RPT // 128):
        buf = rp if p % 2 == 0 else rq
        nbuf = rq if p % 2 == 0 else rp
        sem = gsa if p % 2 == 0 else gsb
        pltpu.async_copy(buf, acc_out.at[c, pl.ds(s * RPT + p * 128, 128)],
                         sem)
        if p + 1 < RPT // 128:
            pltpu.sync_copy(
                acc_sh.at[pl.ds(s * RPT + (p + 1) * 128, 128)], nbuf)
        pltpu.make_async_copy(
            buf, acc_out.at[c, pl.ds(s * RPT, 128)], sem).wait()


@functools.partial(
    pl.kernel,
    out_type=jax.ShapeDtypeStruct((2, 16, N_PAD), jnp.float32),
    mesh=_MESH,
    scratch_types=[
        pltpu.VMEM((RPT, 16), jnp.float32),
        pltpu.VMEM((RPT, 16), jnp.float32),
        pltpu.VMEM((RPT,), jnp.float32),
        pltpu.VMEM((16,), jnp.float32),
        pltpu.VMEM((16, 128), jnp.float32),
        pltpu.VMEM_SHARED((N_PAD, 16), jnp.float32),
        pltpu.VMEM_SHARED((N_PAD, 16), jnp.float32),
    ] + _PROP_SCRATCH,
    compiler_params=_SC_PARAMS,
)
def _sc_layer2(src_hbm, dst_hbm, acc1_hbm, dinv_hbm, b1_hbm, acc_out,
               a0_v, a1_v, dinv_v, b1_v, t_v, table_sh, acc_sh,
               sa, da, ra, gsa, isa, sb, db, rb, gsb, isb):
    c = lax.axis_index("c")
    s = lax.axis_index("s")
    w = c * 16 + s

    sl = pl.ds(s * RPT, RPT)
    pltpu.sync_copy(acc1_hbm.at[0, sl], a0_v)
    pltpu.sync_copy(acc1_hbm.at[1, sl], a1_v)
    pltpu.sync_copy(dinv_hbm.at[sl], dinv_v)
    pltpu.sync_copy(b1_hbm, b1_v)
    b1 = b1_v[...]

    def r2row(g, _):
        dv = dinv_v[pl.ds(g * 16, 16)]
        for j in range(16):
            r = g * 16 + j
            t = dv[j] * (a0_v[r, :] + a1_v[r, :]) + b1
            a0_v[r, :] = dv[j] * jnp.maximum(t, 0.0)
        return 0

    lax.fori_loop(0, RPT // 16, r2row, 0)
    pltpu.sync_copy(a0_v, table_sh.at[sl])

    @pl.when(c == 0)
    def _():
        pltpu.sync_copy(a0_v, acc_sh.at[sl])

    @pl.when(c == 1)
    def _():
        _zero_rows(a0_v, RPT)
        pltpu.sync_copy(a0_v, acc_sh.at[sl])

    plsc.subcore_barrier()
    _propagate(w, src_hbm, dst_hbm, table_sh, acc_sh,
               sa, da, ra, gsa, isa, sb, db, rb, gsb, isb)
    plsc.subcore_barrier()

    lanes = lax.iota(jnp.int32, 16)
    rp = ra.at[pl.ds(0, 128)]
    rq = rb.at[pl.ds(0, 128)]
    pltpu.async_copy(acc_sh.at[pl.ds(s * RPT, 128)], rp, gsa)
    for p in range(RPT // 128):
        buf, nbuf, sem, nsem = ((ra, rb, gsa, gsb) if p % 2 == 0
                                else (rb, ra, gsb, gsa))
        pltpu.make_async_copy(
            acc_sh.at[pl.ds(s * RPT, 128)],
            rp if p % 2 == 0 else rq, sem).wait()
        if p + 1 < RPT // 128:
            pltpu.async_copy(acc_sh.at[pl.ds(s * RPT + (p + 1) * 128, 128)],
                             rq if p % 2 == 0 else rp, nsem)

        def tb(r, _):
            v = buf[r, :]
            plsc.store_scatter(t_v, [lanes, jnp.full((16,), r, jnp.int32)], v)
            return 0

        lax.fori_loop(0, 128, tb, 0, unroll=8)
        pltpu.sync_copy(
            t_v, acc_out.at[c, :, pl.ds(s * RPT + p * 128, 128)])


def _tc_mm1_body(x_ref, w1_ref, out_ref):
    out_ref[...] = jnp.dot(x_ref[...], w1_ref[...],
                           preferred_element_type=jnp.float32)


def _tc_mm1(x, w1):
    return pl.pallas_call(
        _tc_mm1_body,
        out_shape=jax.ShapeDtypeStruct((N_NODES, 16), jnp.float32),
    )(x, w1)


def _tc_out_body(acc_ref, dinv_ref, w2_ref, b2_ref, out_ref):
    a = (acc_ref[0] + acc_ref[1]) * dinv_ref[...][None, :]
    z = lax.dot_general(w2_ref[...], a, (((0,), (0,)), ((), ())),
                        preferred_element_type=jnp.float32)
    z = z + b2_ref[...][:, None]
    m = jnp.max(z, axis=0, keepdims=True)
    t = z - m
    out_ref[...] = t - jnp.log(jnp.sum(jnp.exp(t), axis=0, keepdims=True))


def _tc_out(acc, dinv, w2, b2):
    return pl.pallas_call(
        _tc_out_body,
        out_shape=jax.ShapeDtypeStruct((7, N_PAD), jnp.float32),
    )(acc, dinv, w2, b2)


def kernel(x, edge_index, W1, b1, W2, b2):
    ei = edge_index.astype(jnp.int32)
    src, dst = ei[0], ei[1]
    npad = E_PAD - N_EDGES
    pad_idx = N_NODES + jnp.arange(npad, dtype=jnp.int32) % (N_PAD - N_NODES)
    srcp = jnp.concatenate([src, pad_idx]).reshape(32, EPT)
    dstp = jnp.concatenate([dst, pad_idx]).reshape(32, EPT)

    deg = _sc_degree(dstp)
    h1 = _tc_mm1(x, W1)
    h1_pad = jnp.pad(h1, ((0, N_PAD - N_NODES), (0, 0)))
    acc1, dinv = _sc_layer1(srcp, dstp, h1_pad, deg)
    acc2 = _sc_layer2(srcp, dstp, acc1, dinv, b1)
    zt = _tc_out(acc2, dinv, W2, b2)
    return zt[:, :N_NODES].T


# speedup vs baseline: 1.1169x; 1.0110x over previous
import functools

import jax
import jax.numpy as jnp
from jax import lax
from jax.experimental import pallas as pl
from jax.experimental.pallas import tpu as pltpu
from jax.experimental.pallas import tpu_sc as plsc

N_NODES = 10000
N_EDGES = 320000
N_PAD = 10240
E_PAD = 327680
EPT = E_PAD // 32
G = 1024
NG = EPT // G
RPT = N_PAD // 16

_MESH = plsc.VectorSubcoreMesh(core_axis_name="c", subcore_axis_name="s")
_SC_PARAMS = pltpu.CompilerParams(
    use_tc_tiling_on_sc=False, needs_layout_passes=False)


def _rsqrt16(d):
    i = plsc.bitcast(d, jnp.int32)
    y = plsc.bitcast(0x5F3759DF - lax.shift_right_logical(i, 1), jnp.float32)
    for _ in range(3):
        y = y * (1.5 - 0.5 * d * y * y)
    return y


def _zero_rows(ref, n):
    z = jnp.zeros((16,), jnp.float32)

    def body(i, _):
        ref[i, :] = z
        return 0

    lax.fori_loop(0, n, body, 0, unroll=8)


@functools.partial(
    pl.kernel,
    out_type=jax.ShapeDtypeStruct((2, N_PAD), jnp.float32),
    mesh=_MESH,
    scratch_types=[
        pltpu.VMEM((EPT,), jnp.int32),
        pltpu.VMEM((EPT,), jnp.float32),
        pltpu.VMEM((RPT,), jnp.float32),
        pltpu.VMEM_SHARED((N_PAD,), jnp.float32),
    ],
    compiler_params=_SC_PARAMS,
)
def _sc_degree(dst_hbm, out_hbm, dst_v, ones_v, buf_v, deg_sh):
    c = lax.axis_index("c")
    s = lax.axis_index("s")

    one = jnp.ones((16,), jnp.float32)
    z = jnp.zeros((16,), jnp.float32)

    def ob(i, _):
        ones_v[pl.ds(i * 16, 16)] = one
        return 0

    lax.fori_loop(0, EPT // 16, ob, 0, unroll=8)

    def zb(i, _):
        buf_v[pl.ds(i * 16, 16)] = z
        return 0

    lax.fori_loop(0, RPT // 16, zb, 0, unroll=8)
    pltpu.sync_copy(buf_v, deg_sh.at[pl.ds(s * RPT, RPT)])
    plsc.subcore_barrier()

    for half in range(2):
        pltpu.sync_copy(dst_hbm.at[half * 16 + s], dst_v)
        pltpu.sync_copy(ones_v, deg_sh.at[dst_v], add=True)
    plsc.subcore_barrier()
    pltpu.sync_copy(deg_sh.at[pl.ds(s * RPT, RPT)], buf_v)
    pltpu.sync_copy(buf_v, out_hbm.at[c, pl.ds(s * RPT, RPT)])


def _propagate(w, src_hbm, dst_hbm, table_sh, acc_sh,
               sa, da, ra, gsa, isa, sb, db, rb, gsb, isb):
    def load_idx(g, srcb, dstb, isem):
        pltpu.async_copy(src_hbm.at[w, pl.ds(g * G, G)], srcb, isem)
        pltpu.async_copy(dst_hbm.at[w, pl.ds(g * G, G)], dstb, isem)

    def wait_idx(srcb, dstb, isem):
        pltpu.make_async_copy(src_hbm.at[w, pl.ds(0, G)], srcb, isem).wait()
        pltpu.make_async_copy(dst_hbm.at[w, pl.ds(0, G)], dstb, isem).wait()

    def wait_gather(rows, gsem):
        pltpu.make_async_copy(table_sh.at[sa], rows, gsem).wait()

    load_idx(0, sa, da, isa)
    wait_idx(sa, da, isa)
    pltpu.async_copy(table_sh.at[sa], ra, gsa)
    load_idx(1, sb, db, isb)

    def pair(p, _):
        wait_idx(sb, db, isb)
        pltpu.async_copy(table_sh.at[sb], rb, gsb)
        wait_gather(ra, gsa)
        pltpu.sync_copy(ra, acc_sh.at[da], add=True)
        load_idx(2 * p + 2, sa, da, isa)
        wait_gather(rb, gsb)
        pltpu.sync_copy(rb, acc_sh.at[db], add=True)
        load_idx(2 * p + 3, sb, db, isb)
        wait_idx(sa, da, isa)
        pltpu.async_copy(table_sh.at[sa], ra, gsa)
        return 0

    lax.fori_loop(0, NG // 2 - 1, pair, 0)
    wait_idx(sb, db, isb)
    pltpu.async_copy(table_sh.at[sb], rb, gsb)
    wait_gather(ra, gsa)
    pltpu.sync_copy(ra, acc_sh.at[da], add=True)
    wait_gather(rb, gsb)
    pltpu.sync_copy(rb, acc_sh.at[db], add=True)


_PROP_SCRATCH = [
    pltpu.VMEM((G,), jnp.int32),
    pltpu.VMEM((G,), jnp.int32),
    pltpu.VMEM((G, 16), jnp.float32),
    pltpu.SemaphoreType.DMA,
    pltpu.SemaphoreType.DMA,
    pltpu.VMEM((G,), jnp.int32),
    pltpu.VMEM((G,), jnp.int32),
    pltpu.VMEM((G, 16), jnp.float32),
    pltpu.SemaphoreType.DMA,
    pltpu.SemaphoreType.DMA,
]


@functools.partial(
    pl.kernel,
    out_type=(
        jax.ShapeDtypeStruct((2, N_PAD, 16), jnp.float32),
        jax.ShapeDtypeStruct((N_PAD,), jnp.float32),
    ),
    mesh=_MESH,
    scratch_types=[
        pltpu.VMEM((RPT,), jnp.float32),
        pltpu.VMEM((RPT,), jnp.float32),
        pltpu.VMEM((RPT, 16), jnp.float32),
        pltpu.VMEM_SHARED((N_PAD, 16), jnp.float32),
        pltpu.VMEM_SHARED((N_PAD, 16), jnp.float32),
    ] + _PROP_SCRATCH,
    compiler_params=_SC_PARAMS,
)
def _sc_layer1(src_hbm, dst_hbm, h1_hbm, deg_hbm, acc_out, dinv_out,
               deg_v, dinv_v, h1_v, table_sh, acc_sh,
               sa, da, ra, gsa, isa, sb, db, rb, gsb, isb):
    c = lax.axis_index("c")
    s = lax.axis_index("s")
    w = c * 16 + s

    sl = pl.ds(s * RPT, RPT)
    pltpu.sync_copy(deg_hbm.at[c, sl], deg_v)
    pltpu.sync_copy(h1_hbm.at[sl], h1_v)

    def dg(i, _):
        d = deg_v[pl.ds(i * 16, 16)] + 1.0
        dinv_v[pl.ds(i * 16, 16)] = _rsqrt16(d)
        return 0

    lax.fori_loop(0, RPT // 16, dg, 0)

    def rscale(g, _):
        dv = dinv_v[pl.ds(g * 16, 16)]
        for j in range(16):
            r = g * 16 + j
            h1_v[r, :] = h1_v[r, :] * dv[j]
        return 0

    lax.fori_loop(0, RPT // 16, rscale, 0)
    pltpu.sync_copy(h1_v, table_sh.at[sl])

    @pl.when(c == 0)
    def _():
        pltpu.sync_copy(h1_v, acc_sh.at[sl])
        pltpu.sync_copy(dinv_v, dinv_out.at[sl])

    @pl.when(c == 1)
    def _():
        _zero_rows(h1_v, RPT)
        pltpu.sync_copy(h1_v, acc_sh.at[sl])

    plsc.subcore_barrier()
    _propagate(w, src_hbm, dst_hbm, table_sh, acc_sh,
               sa, da, ra, gsa, isa, sb, db, rb, gsb, isb)
    plsc.subcore_barrier()
    rp = ra.at[pl.ds(0, 128)]
    rq = rb.at[pl.ds(0, 128)]
    pltpu.sync_copy(acc_sh.at[pl.ds(s * RPT, 128)], rp)
    for p in range(RPT // 128):
        buf = rp if p % 2 == 0 else rq
        nbuf = rq if p % 2 == 0 else rp
        sem = gsa if p % 2 == 0 else gsb
        pltpu.async_copy(buf, acc_out.at[c, pl.ds(s * RPT + p * 128, 128)],
                         sem)
        if p + 1 < RPT // 128:
            pltpu.sync_copy(
                acc_sh.at[pl.ds(s * RPT + (p + 1) * 128, 128)], nbuf)
        pltpu.make_async_copy(
            buf, acc_out.at[c, pl.ds(s * RPT, 128)], sem).wait()


@functools.partial(
    pl.kernel,
    out_type=jax.ShapeDtypeStruct((2, 16, N_PAD), jnp.float32),
    mesh=_MESH,
    scratch_types=[
        pltpu.VMEM((RPT, 16), jnp.float32),
        pltpu.VMEM((RPT, 16), jnp.float32),
        pltpu.VMEM((RPT,), jnp.float32),
        pltpu.VMEM((16,), jnp.float32),
        pltpu.VMEM((16, 128), jnp.float32),
        pltpu.VMEM_SHARED((N_PAD, 16), jnp.float32),
        pltpu.VMEM_SHARED((N_PAD, 16), jnp.float32),
    ] + _PROP_SCRATCH,
    compiler_params=_SC_PARAMS,
)
def _sc_layer2(src_hbm, dst_hbm, acc1_hbm, dinv_hbm, b1_hbm, acc_out,
               a0_v, a1_v, dinv_v, b1_v, t_v, table_sh, acc_sh,
               sa, da, ra, gsa, isa, sb, db, rb, gsb, isb):
    c = lax.axis_index("c")
    s = lax.axis_index("s")
    w = c * 16 + s

    sl = pl.ds(s * RPT, RPT)
    pltpu.async_copy(acc1_hbm.at[0, sl], a0_v, gsa)
    pltpu.async_copy(acc1_hbm.at[1, sl], a1_v, gsb)
    pltpu.sync_copy(dinv_hbm.at[sl], dinv_v)
    pltpu.sync_copy(b1_hbm, b1_v)
    pltpu.make_async_copy(acc1_hbm.at[0, sl], a0_v, gsa).wait()
    pltpu.make_async_copy(acc1_hbm.at[1, sl], a1_v, gsb).wait()
    b1 = b1_v[...]

    def r2row(g, _):
        dv = dinv_v[pl.ds(g * 16, 16)]
        for j in range(16):
            r = g * 16 + j
            t = dv[j] * (a0_v[r, :] + a1_v[r, :]) + b1
            a0_v[r, :] = dv[j] * jnp.maximum(t, 0.0)
        return 0

    lax.fori_loop(0, RPT // 16, r2row, 0)
    pltpu.sync_copy(a0_v, table_sh.at[sl])

    @pl.when(c == 0)
    def _():
        pltpu.sync_copy(a0_v, acc_sh.at[sl])

    @pl.when(c == 1)
    def _():
        _zero_rows(a0_v, RPT)
        pltpu.sync_copy(a0_v, acc_sh.at[sl])

    plsc.subcore_barrier()
    _propagate(w, src_hbm, dst_hbm, table_sh, acc_sh,
               sa, da, ra, gsa, isa, sb, db, rb, gsb, isb)
    plsc.subcore_barrier()

    lanes = lax.iota(jnp.int32, 16)
    rp = ra.at[pl.ds(0, 128)]
    rq = rb.at[pl.ds(0, 128)]
    pltpu.async_copy(acc_sh.at[pl.ds(s * RPT, 128)], rp, gsa)
    for p in range(RPT // 128):
        buf, nbuf, sem, nsem = ((ra, rb, gsa, gsb) if p % 2 == 0
                                else (rb, ra, gsb, gsa))
        pltpu.make_async_copy(
            acc_sh.at[pl.ds(s * RPT, 128)],
            rp if p % 2 == 0 else rq, sem).wait()
        if p + 1 < RPT // 128:
            pltpu.async_copy(acc_sh.at[pl.ds(s * RPT + (p + 1) * 128, 128)],
                             rq if p % 2 == 0 else rp, nsem)

        def tb(r, _):
            v = buf[r, :]
            plsc.store_scatter(t_v, [lanes, jnp.full((16,), r, jnp.int32)], v)
            return 0

        lax.fori_loop(0, 128, tb, 0, unroll=8)
        pltpu.sync_copy(
            t_v, acc_out.at[c, :, pl.ds(s * RPT + p * 128, 128)])


def _tc_mm1_body(x_ref, w1_ref, out_ref):
    h = jnp.dot(x_ref[...], w1_ref[...], preferred_element_type=jnp.float32)
    out_ref[...] = jnp.pad(h, ((0, N_PAD - N_NODES), (0, 0)))


def _tc_mm1(x, w1):
    return pl.pallas_call(
        _tc_mm1_body,
        out_shape=jax.ShapeDtypeStruct((N_PAD, 16), jnp.float32),
    )(x, w1)


def _tc_out_body(acc_ref, dinv_ref, w2_ref, b2_ref, out_ref):
    a = (acc_ref[0] + acc_ref[1]) * dinv_ref[...][None, :]
    z = lax.dot_general(w2_ref[...], a, (((0,), (0,)), ((), ())),
                        preferred_element_type=jnp.float32)
    z = z + b2_ref[...][:, None]
    m = jnp.max(z, axis=0, keepdims=True)
    t = z - m
    out_ref[...] = t - jnp.log(jnp.sum(jnp.exp(t), axis=0, keepdims=True))


def _tc_out(acc, dinv, w2, b2):
    return pl.pallas_call(
        _tc_out_body,
        out_shape=jax.ShapeDtypeStruct((7, N_PAD), jnp.float32),
    )(acc, dinv, w2, b2)


def kernel(x, edge_index, W1, b1, W2, b2):
    ei = edge_index.astype(jnp.int32)
    src, dst = ei[0], ei[1]
    npad = E_PAD - N_EDGES
    pad_idx = N_NODES + jnp.arange(npad, dtype=jnp.int32) % (N_PAD - N_NODES)
    srcp = jnp.concatenate([src, pad_idx]).reshape(32, EPT)
    dstp = jnp.concatenate([dst, pad_idx]).reshape(32, EPT)

    deg = _sc_degree(dstp)
    h1_pad = _tc_mm1(x, W1)
    acc1, dinv = _sc_layer1(srcp, dstp, h1_pad, deg)
    acc2 = _sc_layer2(srcp, dstp, acc1, dinv, b1)
    zt = _tc_out(acc2, dinv, W2, b2)
    return zt[:, :N_NODES].T
